# baseline (speedup 1.0000x reference)
import jax
import jax.numpy as jnp
from jax import lax
from jax.experimental import pallas as pl
from jax.experimental.pallas import tpu as pltpu
from jax.experimental.pallas import tpu_sc as plsc

NUM_KNOTS = 30
IN_DIM = 64
OUT_DIM = 64
N_SAMPLES = 8192
ETA = 1.0 / (NUM_KNOTS - 1)
PHI_MAX = 1.0 + (OUT_DIM - 1) * ETA
DPHI = PHI_MAX / (NUM_KNOTS - 1)
DPHI2 = 1.0 / (NUM_KNOTS - 1)
NBINS = 896
HW_COLS = 2 * NBINS
LANE_BLK = 128
NWORKERS = 32
ROWS_PER_W = N_SAMPLES // NWORKERS
XCHUNK = 128
GRP = 16


def _sc_hist_body(xt_hbm, lamb_hbm, out_hbm):
    def _inner(lam_v, x_v, h0, h1, s0, s1):
        wid = lax.axis_index("s") * 2 + lax.axis_index("c")
        lanes = lax.iota(jnp.int32, 16)
        pltpu.sync_copy(lamb_hbm, lam_v)

        copies = [None, None]
        gg = 0
        for blk in range(ROWS_PER_W // XCHUNK):
            colbase = wid * ROWS_PER_W + blk * XCHUNK
            pltpu.sync_copy(
                xt_hbm.at[:, pl.ds(pl.multiple_of(colbase, 128), XCHUNK)], x_v)
            for g in range(XCHUNK // GRP):
                buf = h0 if gg % 2 == 0 else h1
                sem = s0 if gg % 2 == 0 else s1
                if gg >= 2:
                    copies[gg % 2].wait()

                def zrow(r, _, buf=buf):
                    def zcol(jj, __):
                        for k in range(8):
                            buf[r, jj, pl.ds(k * 16, 16)] = (
                                jnp.zeros((16,), jnp.float32))
                        return 0
                    lax.fori_loop(0, HW_COLS // 128, zcol, 0)
                    return 0
                lax.fori_loop(0, GRP, zrow, 0)

                def pbody(p, _, buf=buf, g=g):
                    xr = x_v[p, pl.ds(g * GRP, GRP)]
                    lr = lam_v[p]
                    nb = jnp.minimum((xr * 841.0).astype(jnp.int32), 840)
                    plsc.addupdate_scatter(
                        buf, [lanes, nb >> 7, nb & 127], lr)
                    plsc.addupdate_scatter(
                        buf, [lanes, (nb >> 7) + (NBINS // 128), nb & 127],
                        lr * xr)
                    return 0
                lax.fori_loop(0, IN_DIM, pbody, 0)

                rowbase = colbase + g * GRP
                copies[gg % 2] = pltpu.async_copy(
                    buf,
                    out_hbm.at[pl.ds(pl.multiple_of(rowbase, 8), GRP), :, :],
                    sem)
                gg += 1
        copies[0].wait()
        copies[1].wait()

    pl.run_scoped(
        _inner,
        pltpu.VMEM((IN_DIM, 16), jnp.float32),
        pltpu.VMEM((IN_DIM, XCHUNK), jnp.float32),
        pltpu.VMEM((GRP, HW_COLS // 128, 128), jnp.float32),
        pltpu.VMEM((GRP, HW_COLS // 128, 128), jnp.float32),
        pltpu.SemaphoreType.DMA,
        pltpu.SemaphoreType.DMA,
    )


def _sc_hist(xt, lamb16):
    mesh = plsc.VectorSubcoreMesh(core_axis_name="c", subcore_axis_name="s")
    return pl.kernel(
        _sc_hist_body,
        mesh=mesh,
        out_type=jax.ShapeDtypeStruct((N_SAMPLES, HW_COLS // 128, 128),
                                      jnp.float32),
        compiler_params=pltpu.CompilerParams(needs_layout_passes=False),
    )(xt, lamb16)


def _tc_eval_kernel(hw_ref, pli_ref, pc_ref, ccr_ref, out_ref, tc_ref, tb_ref):
    inc = jax.nn.softplus(pli_ref[...])
    tot = jnp.sum(inc) + 1e-8
    c0 = inc[0, 0] / tot
    minv = 1.0 / (tot * DPHI)
    m = [inc[0, j + 1] * minv for j in range(NUM_KNOTS - 1)]
    A = {i: m[i - 1] - m[i] for i in range(1, NUM_KNOTS - 1)}
    A[NUM_KNOTS - 1] = m[NUM_KNOTS - 2]

    @pl.when(pl.program_id(0) == 0)
    def _build_tables():
        ni = lax.broadcasted_iota(jnp.int32, (NBINS, OUT_DIM), 0).astype(
            jnp.float32)
        qi = lax.broadcasted_iota(jnp.int32, (NBINS, OUT_DIM), 1).astype(
            jnp.float32)
        zf = qi * 29.0 + ni
        bacc = jnp.zeros((NBINS, OUT_DIM), jnp.float32)
        cacc = jnp.zeros((NBINS, OUT_DIM), jnp.float32)
        for i in range(1, NUM_KNOTS):
            hi = zf < float(92 * i)
            bacc = bacc + jnp.where(hi, A[i], 0.0)
            cacc = cacc + jnp.where(hi, 0.0, A[i] * float(i * DPHI))
        tb_ref[...] = bacc
        tc_ref[...] = cacc + c0 + (qi * ETA) * bacc

    s = (lax.dot_general(hw_ref[:, 0:NBINS], tc_ref[...],
                         (((1,), (0,)), ((), ())),
                         precision=lax.Precision.HIGHEST,
                         preferred_element_type=jnp.float32)
         + lax.dot_general(hw_ref[:, NBINS:HW_COLS], tb_ref[...],
                           (((1,), (0,)), ((), ())),
                           precision=lax.Precision.HIGHEST,
                           preferred_element_type=jnp.float32))

    C = pc_ref[...]
    cmin = jnp.min(C)
    cmax = jnp.max(C)
    cc = ccr_ref[0, 0]
    cr = ccr_ref[0, 1]
    alpha = 2.0 * cr / (cmax - cmin + 1e-8)
    tc0 = cc - cr + alpha * (C[0, 0] - cmin)
    M2 = [alpha * (C[0, j + 1] - C[0, j]) / DPHI2 for j in range(NUM_KNOTS - 1)]
    A2 = [M2[i - 1] - M2[i] for i in range(1, NUM_KNOTS - 1)] + [M2[NUM_KNOTS - 2]]

    Sc = jnp.maximum(s, 0.0)
    y = A2[0] * jnp.minimum(Sc, DPHI2)
    for i in range(1, NUM_KNOTS - 1):
        y = y + A2[i] * jnp.minimum(Sc, (i + 1) * DPHI2)
    out_ref[...] = y + tc0


def _tc_eval(hw, pli2, pc2, ccr):
    grid = (N_SAMPLES // LANE_BLK,)
    return pl.pallas_call(
        _tc_eval_kernel,
        grid=grid,
        in_specs=[
            pl.BlockSpec((LANE_BLK, HW_COLS), lambda i: (i, 0)),
            pl.BlockSpec((1, NUM_KNOTS), lambda i: (0, 0)),
            pl.BlockSpec((1, NUM_KNOTS), lambda i: (0, 0)),
            pl.BlockSpec((1, 2), lambda i: (0, 0)),
        ],
        out_specs=pl.BlockSpec((LANE_BLK, OUT_DIM), lambda i: (i, 0)),
        out_shape=jax.ShapeDtypeStruct((N_SAMPLES, OUT_DIM), jnp.float32),
        scratch_shapes=[
            pltpu.VMEM((NBINS, OUT_DIM), jnp.float32),
            pltpu.VMEM((NBINS, OUT_DIM), jnp.float32),
        ],
    )(hw, pli2, pc2, ccr)


def kernel(x, phi_log_increments, Phi_coeffs, lambdas, cc, cr):
    xt = x.T
    pli2 = phi_log_increments.reshape(1, NUM_KNOTS)
    pc2 = Phi_coeffs.reshape(1, NUM_KNOTS)
    ccr = jnp.stack([jnp.asarray(cc, jnp.float32),
                     jnp.asarray(cr, jnp.float32)]).reshape(1, 2)
    lamb16 = jnp.broadcast_to(lambdas.reshape(IN_DIM, 1), (IN_DIM, 16))

    hw3 = _sc_hist(xt, lamb16)
    hw = hw3.reshape(N_SAMPLES, HW_COLS)
    return _tc_eval(hw, pli2, pc2, ccr)


# speedup vs baseline: 1.0602x; 1.0602x over previous
import jax
import jax.numpy as jnp
from jax import lax
from jax.experimental import pallas as pl
from jax.experimental.pallas import tpu as pltpu
from jax.experimental.pallas import tpu_sc as plsc

NUM_KNOTS = 30
IN_DIM = 64
OUT_DIM = 64
N_SAMPLES = 8192
ETA = 1.0 / (NUM_KNOTS - 1)
PHI_MAX = 1.0 + (OUT_DIM - 1) * ETA
DPHI = PHI_MAX / (NUM_KNOTS - 1)
DPHI2 = 1.0 / (NUM_KNOTS - 1)
NBINS = 896
HW_COLS = 2 * NBINS
LANE_BLK = 128
NWORKERS = 32
ROWS_PER_W = N_SAMPLES // NWORKERS
XCHUNK = 128
GRP = 16


def _sc_hist_body(xt_hbm, lamb_hbm, out_hbm):
    def _inner(lam_v, x_v, h0, h1, s0, s1):
        wid = lax.axis_index("s") * 2 + lax.axis_index("c")
        lanes = lax.iota(jnp.int32, 16)
        pltpu.sync_copy(lamb_hbm, lam_v)

        copies = [None, None]
        gg = 0
        for blk in range(ROWS_PER_W // XCHUNK):
            colbase = wid * ROWS_PER_W + blk * XCHUNK
            pltpu.sync_copy(
                xt_hbm.at[:, pl.ds(pl.multiple_of(colbase, 128), XCHUNK)], x_v)
            for g in range(XCHUNK // GRP):
                buf = h0 if gg % 2 == 0 else h1
                sem = s0 if gg % 2 == 0 else s1
                if gg >= 2:
                    copies[gg % 2].wait()

                def zrow(r, _, buf=buf):
                    def zcol(jj, __):
                        for k in range(8):
                            buf[r, jj, pl.ds(k * 16, 16)] = (
                                jnp.zeros((16,), jnp.float32))
                        return 0
                    lax.fori_loop(0, HW_COLS // 128, zcol, 0)
                    return 0
                lax.fori_loop(0, GRP, zrow, 0)

                def pbody(p, _, buf=buf, g=g):
                    xr = x_v[p, pl.ds(g * GRP, GRP)]
                    lr = lam_v[p]
                    nb = jnp.minimum((xr * 841.0).astype(jnp.int32), 840)
                    plsc.addupdate_scatter(
                        buf, [lanes, nb >> 7, nb & 127], lr)
                    plsc.addupdate_scatter(
                        buf, [lanes, (nb >> 7) + (NBINS // 128), nb & 127],
                        lr * xr)
                    return 0
                lax.fori_loop(0, IN_DIM, pbody, 0)

                rowbase = colbase + g * GRP
                copies[gg % 2] = pltpu.async_copy(
                    buf,
                    out_hbm.at[pl.ds(pl.multiple_of(rowbase, 8), GRP), :, :],
                    sem)
                gg += 1
        copies[0].wait()
        copies[1].wait()

    pl.run_scoped(
        _inner,
        pltpu.VMEM((IN_DIM, 16), jnp.float32),
        pltpu.VMEM((IN_DIM, XCHUNK), jnp.float32),
        pltpu.VMEM((GRP, HW_COLS // 128, 128), jnp.float32),
        pltpu.VMEM((GRP, HW_COLS // 128, 128), jnp.float32),
        pltpu.SemaphoreType.DMA,
        pltpu.SemaphoreType.DMA,
    )


def _sc_hist(xt, lamb16):
    mesh = plsc.VectorSubcoreMesh(core_axis_name="c", subcore_axis_name="s")
    return pl.kernel(
        _sc_hist_body,
        mesh=mesh,
        out_type=jax.ShapeDtypeStruct((N_SAMPLES, HW_COLS // 128, 128),
                                      jnp.float32),
        compiler_params=pltpu.CompilerParams(needs_layout_passes=False),
    )(xt, lamb16)


def _tc_table_kernel(pli_ref, t_ref):
    inc = jax.nn.softplus(pli_ref[...])
    tot = jnp.sum(inc) + 1e-8
    c0 = inc[0, 0] / tot
    minv = 1.0 / (tot * DPHI)
    m = [inc[0, j + 1] * minv for j in range(NUM_KNOTS - 1)]
    A = {i: m[i - 1] - m[i] for i in range(1, NUM_KNOTS - 1)}
    A[NUM_KNOTS - 1] = m[NUM_KNOTS - 2]

    ni = lax.broadcasted_iota(jnp.int32, (NBINS, OUT_DIM), 0).astype(
        jnp.float32)
    qi = lax.broadcasted_iota(jnp.int32, (NBINS, OUT_DIM), 1).astype(
        jnp.float32)
    zf = qi * 29.0 + ni
    bacc = jnp.zeros((NBINS, OUT_DIM), jnp.float32)
    cacc = jnp.zeros((NBINS, OUT_DIM), jnp.float32)
    for i in range(1, NUM_KNOTS):
        hi = zf < float(92 * i)
        bacc = bacc + jnp.where(hi, A[i], 0.0)
        cacc = cacc + jnp.where(hi, 0.0, A[i] * float(i * DPHI))
    t_ref[0:NBINS, :] = cacc + c0 + (qi * ETA) * bacc
    t_ref[NBINS:HW_COLS, :] = bacc


def _tc_table(pli2):
    return pl.pallas_call(
        _tc_table_kernel,
        out_shape=jax.ShapeDtypeStruct((HW_COLS, OUT_DIM), jnp.float32),
    )(pli2)


def _tc_eval_kernel(hw_ref, t_ref, pc_ref, ccr_ref, out_ref):
    s = lax.dot_general(hw_ref[...], t_ref[...],
                        (((1,), (0,)), ((), ())),
                        precision=lax.Precision.HIGHEST,
                        preferred_element_type=jnp.float32)

    C = pc_ref[...]
    cmin = jnp.min(C)
    cmax = jnp.max(C)
    cc = ccr_ref[0, 0]
    cr = ccr_ref[0, 1]
    alpha = 2.0 * cr / (cmax - cmin + 1e-8)
    tc0 = cc - cr + alpha * (C[0, 0] - cmin)
    M2 = [alpha * (C[0, j + 1] - C[0, j]) / DPHI2 for j in range(NUM_KNOTS - 1)]
    A2 = [M2[i - 1] - M2[i] for i in range(1, NUM_KNOTS - 1)] + [M2[NUM_KNOTS - 2]]

    Sc = jnp.maximum(s, 0.0)
    y = A2[0] * jnp.minimum(Sc, DPHI2)
    for i in range(1, NUM_KNOTS - 1):
        y = y + A2[i] * jnp.minimum(Sc, (i + 1) * DPHI2)
    out_ref[...] = y + tc0


def _tc_eval(hw, tbl, pc2, ccr):
    grid = (N_SAMPLES // LANE_BLK,)
    return pl.pallas_call(
        _tc_eval_kernel,
        grid=grid,
        in_specs=[
            pl.BlockSpec((LANE_BLK, HW_COLS), lambda i: (i, 0)),
            pl.BlockSpec((HW_COLS, OUT_DIM), lambda i: (0, 0)),
            pl.BlockSpec((1, NUM_KNOTS), lambda i: (0, 0)),
            pl.BlockSpec((1, 2), lambda i: (0, 0)),
        ],
        out_specs=pl.BlockSpec((LANE_BLK, OUT_DIM), lambda i: (i, 0)),
        out_shape=jax.ShapeDtypeStruct((N_SAMPLES, OUT_DIM), jnp.float32),
    )(hw, tbl, pc2, ccr)


def kernel(x, phi_log_increments, Phi_coeffs, lambdas, cc, cr):
    xt = x.T
    pli2 = phi_log_increments.reshape(1, NUM_KNOTS)
    pc2 = Phi_coeffs.reshape(1, NUM_KNOTS)
    ccr = jnp.stack([jnp.asarray(cc, jnp.float32),
                     jnp.asarray(cr, jnp.float32)]).reshape(1, 2)
    lamb16 = jnp.broadcast_to(lambdas.reshape(IN_DIM, 1), (IN_DIM, 16))

    hw3 = _sc_hist(xt, lamb16)
    hw = hw3.reshape(N_SAMPLES, HW_COLS)
    tbl = _tc_table(pli2)
    return _tc_eval(hw, tbl, pc2, ccr)


# speedup vs baseline: 1.1130x; 1.0498x over previous
import jax
import jax.numpy as jnp
from jax import lax
from jax.experimental import pallas as pl
from jax.experimental.pallas import tpu as pltpu
from jax.experimental.pallas import tpu_sc as plsc

NUM_KNOTS = 30
IN_DIM = 64
OUT_DIM = 64
N_SAMPLES = 8192
ETA = 1.0 / (NUM_KNOTS - 1)
PHI_MAX = 1.0 + (OUT_DIM - 1) * ETA
DPHI = PHI_MAX / (NUM_KNOTS - 1)
DPHI2 = 1.0 / (NUM_KNOTS - 1)
NBINS = 896
HW_COLS = 2 * NBINS
LANE_BLK = 256
NWORKERS = 32
ROWS_PER_W = N_SAMPLES // NWORKERS
XCHUNK = 128
GRP = 16


def _sc_hist_body(xt_hbm, lamb_hbm, out_hbm):
    def _inner(lam_v, x_v, h0, h1, s0, s1):
        wid = lax.axis_index("s") * 2 + lax.axis_index("c")
        lanes = lax.iota(jnp.int32, 16)
        pltpu.sync_copy(lamb_hbm, lam_v)

        copies = [None, None]
        gg = 0
        for blk in range(ROWS_PER_W // XCHUNK):
            colbase = wid * ROWS_PER_W + blk * XCHUNK
            pltpu.sync_copy(
                xt_hbm.at[:, pl.ds(pl.multiple_of(colbase, 128), XCHUNK)], x_v)
            for g in range(XCHUNK // GRP):
                buf = h0 if gg % 2 == 0 else h1
                sem = s0 if gg % 2 == 0 else s1
                if gg >= 2:
                    copies[gg % 2].wait()

                def zrow(r, _, buf=buf):
                    def zcol(jj, __):
                        for k in range(8):
                            buf[r, jj, pl.ds(k * 16, 16)] = (
                                jnp.zeros((16,), jnp.float32))
                        return 0
                    lax.fori_loop(0, HW_COLS // 128, zcol, 0)
                    return 0
                lax.fori_loop(0, GRP, zrow, 0)

                def pbody(p, _, buf=buf, g=g):
                    xr = x_v[p, pl.ds(g * GRP, GRP)]
                    lr = lam_v[p]
                    nb = jnp.minimum((xr * 841.0).astype(jnp.int32), 840)
                    plsc.addupdate_scatter(
                        buf, [lanes, nb >> 7, nb & 127], lr)
                    plsc.addupdate_scatter(
                        buf, [lanes, (nb >> 7) + (NBINS // 128), nb & 127],
                        lr * xr)
                    return 0
                lax.fori_loop(0, IN_DIM, pbody, 0)

                rowbase = colbase + g * GRP
                copies[gg % 2] = pltpu.async_copy(
                    buf,
                    out_hbm.at[pl.ds(pl.multiple_of(rowbase, 8), GRP), :, :],
                    sem)
                gg += 1
        copies[0].wait()
        copies[1].wait()

    pl.run_scoped(
        _inner,
        pltpu.VMEM((IN_DIM, 16), jnp.float32),
        pltpu.VMEM((IN_DIM, XCHUNK), jnp.float32),
        pltpu.VMEM((GRP, HW_COLS // 128, 128), jnp.float32),
        pltpu.VMEM((GRP, HW_COLS // 128, 128), jnp.float32),
        pltpu.SemaphoreType.DMA,
        pltpu.SemaphoreType.DMA,
    )


def _sc_hist(xt, lamb16):
    mesh = plsc.VectorSubcoreMesh(core_axis_name="c", subcore_axis_name="s")
    return pl.kernel(
        _sc_hist_body,
        mesh=mesh,
        out_type=jax.ShapeDtypeStruct((N_SAMPLES, HW_COLS // 128, 128),
                                      jnp.float32),
        compiler_params=pltpu.CompilerParams(needs_layout_passes=False),
    )(xt, lamb16)


def _tc_table_kernel(pli_ref, t_ref):
    inc = jax.nn.softplus(pli_ref[...])
    tot = jnp.sum(inc) + 1e-8
    c0 = inc[0, 0] / tot
    minv = 1.0 / (tot * DPHI)
    m = [inc[0, j + 1] * minv for j in range(NUM_KNOTS - 1)]
    A = {i: m[i - 1] - m[i] for i in range(1, NUM_KNOTS - 1)}
    A[NUM_KNOTS - 1] = m[NUM_KNOTS - 2]

    ni = lax.broadcasted_iota(jnp.int32, (NBINS, OUT_DIM), 0).astype(
        jnp.float32)
    qi = lax.broadcasted_iota(jnp.int32, (NBINS, OUT_DIM), 1).astype(
        jnp.float32)
    zf = qi * 29.0 + ni
    bacc = jnp.zeros((NBINS, OUT_DIM), jnp.float32)
    cacc = jnp.zeros((NBINS, OUT_DIM), jnp.float32)
    for i in range(1, NUM_KNOTS):
        hi = zf < float(92 * i)
        bacc = bacc + jnp.where(hi, A[i], 0.0)
        cacc = cacc + jnp.where(hi, 0.0, A[i] * float(i * DPHI))
    t_ref[0:NBINS, :] = cacc + c0 + (qi * ETA) * bacc
    t_ref[NBINS:HW_COLS, :] = bacc


def _tc_table(pli2):
    return pl.pallas_call(
        _tc_table_kernel,
        out_shape=jax.ShapeDtypeStruct((HW_COLS, OUT_DIM), jnp.float32),
    )(pli2)


def _tc_eval_kernel(hw_ref, t_ref, pc_ref, ccr_ref, out_ref):
    s = lax.dot_general(hw_ref[...], t_ref[...],
                        (((1,), (0,)), ((), ())),
                        precision=lax.Precision.HIGHEST,
                        preferred_element_type=jnp.float32)

    C = pc_ref[...]
    cmin = jnp.min(C)
    cmax = jnp.max(C)
    cc = ccr_ref[0, 0]
    cr = ccr_ref[0, 1]
    alpha = 2.0 * cr / (cmax - cmin + 1e-8)
    tc0 = cc - cr + alpha * (C[0, 0] - cmin)
    M2 = [alpha * (C[0, j + 1] - C[0, j]) / DPHI2 for j in range(NUM_KNOTS - 1)]
    A2 = [M2[i - 1] - M2[i] for i in range(1, NUM_KNOTS - 1)] + [M2[NUM_KNOTS - 2]]

    Sc = jnp.maximum(s, 0.0)
    y = A2[0] * jnp.minimum(Sc, DPHI2)
    for i in range(1, NUM_KNOTS - 1):
        y = y + A2[i] * jnp.minimum(Sc, (i + 1) * DPHI2)
    out_ref[...] = y + tc0


def _tc_eval(hw, tbl, pc2, ccr):
    grid = (N_SAMPLES // LANE_BLK,)
    return pl.pallas_call(
        _tc_eval_kernel,
        grid=grid,
        in_specs=[
            pl.BlockSpec((LANE_BLK, HW_COLS), lambda i: (i, 0)),
            pl.BlockSpec((HW_COLS, OUT_DIM), lambda i: (0, 0)),
            pl.BlockSpec((1, NUM_KNOTS), lambda i: (0, 0)),
            pl.BlockSpec((1, 2), lambda i: (0, 0)),
        ],
        out_specs=pl.BlockSpec((LANE_BLK, OUT_DIM), lambda i: (i, 0)),
        out_shape=jax.ShapeDtypeStruct((N_SAMPLES, OUT_DIM), jnp.float32),
    )(hw, tbl, pc2, ccr)


def kernel(x, phi_log_increments, Phi_coeffs, lambdas, cc, cr):
    xt = x.T
    pli2 = phi_log_increments.reshape(1, NUM_KNOTS)
    pc2 = Phi_coeffs.reshape(1, NUM_KNOTS)
    ccr = jnp.stack([jnp.asarray(cc, jnp.float32),
                     jnp.asarray(cr, jnp.float32)]).reshape(1, 2)
    lamb16 = jnp.broadcast_to(lambdas.reshape(IN_DIM, 1), (IN_DIM, 16))

    hw3 = _sc_hist(xt, lamb16)
    hw = hw3.reshape(N_SAMPLES, HW_COLS)
    tbl = _tc_table(pli2)
    return _tc_eval(hw, tbl, pc2, ccr)


# speedup vs baseline: 1.2245x; 1.1002x over previous
import jax
import jax.numpy as jnp
from jax import lax
from jax.experimental import pallas as pl
from jax.experimental.pallas import tpu as pltpu
from jax.experimental.pallas import tpu_sc as plsc

NUM_KNOTS = 30
IN_DIM = 64
OUT_DIM = 64
N_SAMPLES = 8192
ETA = 1.0 / (NUM_KNOTS - 1)
PHI_MAX = 1.0 + (OUT_DIM - 1) * ETA
DPHI = PHI_MAX / (NUM_KNOTS - 1)
DPHI2 = 1.0 / (NUM_KNOTS - 1)
NBINS = 896
HW_COLS = 2 * NBINS
LANE_BLK = 256
NWORKERS = 32
ROWS_PER_W = N_SAMPLES // NWORKERS
XCHUNK = 128
GRP = 16


def _sc_hist_body(xt_hbm, lamb_hbm, out_hbm):
    def _inner(lam_v, x_v, h0, h1, s0, s1):
        wid = lax.axis_index("s") * 2 + lax.axis_index("c")
        lanes = lax.iota(jnp.int32, 16)
        pltpu.sync_copy(lamb_hbm, lam_v)

        copies = [None, None]
        gg = 0
        for blk in range(ROWS_PER_W // XCHUNK):
            colbase = wid * ROWS_PER_W + blk * XCHUNK
            pltpu.sync_copy(
                xt_hbm.at[:, pl.ds(pl.multiple_of(colbase, 128), XCHUNK)], x_v)
            for g in range(XCHUNK // GRP):
                buf = h0 if gg % 2 == 0 else h1
                sem = s0 if gg % 2 == 0 else s1
                if gg >= 2:
                    copies[gg % 2].wait()

                def zrow(r, _, buf=buf):
                    def zcol(jj, __):
                        buf[r, pl.ds(jj * 16, 16)] = (
                            jnp.zeros((16,), jnp.float32))
                        return 0
                    lax.fori_loop(0, HW_COLS // 16, zcol, 0)
                    return 0
                lax.fori_loop(0, GRP, zrow, 0)

                def pbody(p, _, buf=buf, g=g):
                    xr = x_v[p, pl.ds(g * GRP, GRP)]
                    lr = lam_v[p]
                    nb = jnp.minimum((xr * 841.0).astype(jnp.int32), 840)
                    plsc.addupdate_scatter(buf, [lanes, nb], lr)
                    plsc.addupdate_scatter(buf, [lanes, nb + NBINS], lr * xr)
                    return 0
                lax.fori_loop(0, IN_DIM, pbody, 0)

                rowbase = colbase + g * GRP
                copies[gg % 2] = pltpu.async_copy(
                    buf,
                    out_hbm.at[pl.ds(pl.multiple_of(rowbase, 8), GRP), :],
                    sem)
                gg += 1
        copies[0].wait()
        copies[1].wait()

    pl.run_scoped(
        _inner,
        pltpu.VMEM((IN_DIM, 16), jnp.float32),
        pltpu.VMEM((IN_DIM, XCHUNK), jnp.float32),
        pltpu.VMEM((GRP, HW_COLS), jnp.float32),
        pltpu.VMEM((GRP, HW_COLS), jnp.float32),
        pltpu.SemaphoreType.DMA,
        pltpu.SemaphoreType.DMA,
    )


def _sc_hist(xt, lamb16):
    mesh = plsc.VectorSubcoreMesh(core_axis_name="c", subcore_axis_name="s")
    return pl.kernel(
        _sc_hist_body,
        mesh=mesh,
        out_type=jax.ShapeDtypeStruct((N_SAMPLES, HW_COLS), jnp.float32),
        compiler_params=pltpu.CompilerParams(needs_layout_passes=False),
    )(xt, lamb16)


def _tc_table_kernel(pli_ref, t_ref):
    inc = jax.nn.softplus(pli_ref[...])
    tot = jnp.sum(inc) + 1e-8
    c0 = inc[0, 0] / tot
    minv = 1.0 / (tot * DPHI)
    m = [inc[0, j + 1] * minv for j in range(NUM_KNOTS - 1)]
    A = {i: m[i - 1] - m[i] for i in range(1, NUM_KNOTS - 1)}
    A[NUM_KNOTS - 1] = m[NUM_KNOTS - 2]

    ni = lax.broadcasted_iota(jnp.int32, (NBINS, OUT_DIM), 0).astype(
        jnp.float32)
    qi = lax.broadcasted_iota(jnp.int32, (NBINS, OUT_DIM), 1).astype(
        jnp.float32)
    zf = qi * 29.0 + ni
    bacc = jnp.zeros((NBINS, OUT_DIM), jnp.float32)
    cacc = jnp.zeros((NBINS, OUT_DIM), jnp.float32)
    for i in range(1, NUM_KNOTS):
        hi = zf < float(92 * i)
        bacc = bacc + jnp.where(hi, A[i], 0.0)
        cacc = cacc + jnp.where(hi, 0.0, A[i] * float(i * DPHI))
    t_ref[0:NBINS, :] = cacc + c0 + (qi * ETA) * bacc
    t_ref[NBINS:HW_COLS, :] = bacc


def _tc_table(pli2):
    return pl.pallas_call(
        _tc_table_kernel,
        out_shape=jax.ShapeDtypeStruct((HW_COLS, OUT_DIM), jnp.float32),
    )(pli2)


def _tc_eval_kernel(hw_ref, t_ref, pc_ref, ccr_ref, out_ref):
    s = lax.dot_general(hw_ref[...], t_ref[...],
                        (((1,), (0,)), ((), ())),
                        precision=lax.Precision.HIGHEST,
                        preferred_element_type=jnp.float32)

    C = pc_ref[...]
    cmin = jnp.min(C)
    cmax = jnp.max(C)
    cc = ccr_ref[0, 0]
    cr = ccr_ref[0, 1]
    alpha = 2.0 * cr / (cmax - cmin + 1e-8)
    tc0 = cc - cr + alpha * (C[0, 0] - cmin)
    M2 = [alpha * (C[0, j + 1] - C[0, j]) / DPHI2 for j in range(NUM_KNOTS - 1)]
    A2 = [M2[i - 1] - M2[i] for i in range(1, NUM_KNOTS - 1)] + [M2[NUM_KNOTS - 2]]

    Sc = jnp.maximum(s, 0.0)
    y = A2[0] * jnp.minimum(Sc, DPHI2)
    for i in range(1, NUM_KNOTS - 1):
        y = y + A2[i] * jnp.minimum(Sc, (i + 1) * DPHI2)
    out_ref[...] = y + tc0


def _tc_eval(hw, tbl, pc2, ccr):
    grid = (N_SAMPLES // LANE_BLK,)
    return pl.pallas_call(
        _tc_eval_kernel,
        grid=grid,
        in_specs=[
            pl.BlockSpec((LANE_BLK, HW_COLS), lambda i: (i, 0)),
            pl.BlockSpec((HW_COLS, OUT_DIM), lambda i: (0, 0)),
            pl.BlockSpec((1, NUM_KNOTS), lambda i: (0, 0)),
            pl.BlockSpec((1, 2), lambda i: (0, 0)),
        ],
        out_specs=pl.BlockSpec((LANE_BLK, OUT_DIM), lambda i: (i, 0)),
        out_shape=jax.ShapeDtypeStruct((N_SAMPLES, OUT_DIM), jnp.float32),
    )(hw, tbl, pc2, ccr)


def kernel(x, phi_log_increments, Phi_coeffs, lambdas, cc, cr):
    xt = x.T
    pli2 = phi_log_increments.reshape(1, NUM_KNOTS)
    pc2 = Phi_coeffs.reshape(1, NUM_KNOTS)
    ccr = jnp.stack([jnp.asarray(cc, jnp.float32),
                     jnp.asarray(cr, jnp.float32)]).reshape(1, 2)
    lamb16 = jnp.broadcast_to(lambdas.reshape(IN_DIM, 1), (IN_DIM, 16))

    hw = _sc_hist(xt, lamb16)
    tbl = _tc_table(pli2)
    return _tc_eval(hw, tbl, pc2, ccr)


# speedup vs baseline: 2.0804x; 1.6990x over previous
import jax
import jax.numpy as jnp
from jax import lax
from jax.experimental import pallas as pl
from jax.experimental.pallas import tpu as pltpu
from jax.experimental.pallas import tpu_sc as plsc

NUM_KNOTS = 30
IN_DIM = 64
OUT_DIM = 64
N_SAMPLES = 8192
ETA = 1.0 / (NUM_KNOTS - 1)
PHI_MAX = 1.0 + (OUT_DIM - 1) * ETA
DPHI = PHI_MAX / (NUM_KNOTS - 1)
DPHI2 = 1.0 / (NUM_KNOTS - 1)
NBINS = 896
HW_COLS = 2 * NBINS
LANE_BLK = 256
NWORKERS = 32
ROWS_PER_W = N_SAMPLES // NWORKERS
XCHUNK = 128
GRP = 16


def _sc_hist_body(xt_hbm, lamb_hbm, out_hbm):
    def _inner(lam_v, x_v, h0, h1, s0, s1):
        wid = lax.axis_index("s") * 2 + lax.axis_index("c")
        lanes = lax.iota(jnp.int32, 16)
        pltpu.sync_copy(lamb_hbm, lam_v)

        copies = [None, None]
        gg = 0
        for blk in range(ROWS_PER_W // XCHUNK):
            colbase = wid * ROWS_PER_W + blk * XCHUNK
            pltpu.sync_copy(
                xt_hbm.at[:, pl.ds(pl.multiple_of(colbase, 128), XCHUNK)], x_v)
            for g in range(XCHUNK // GRP):
                buf = h0 if gg % 2 == 0 else h1
                sem = s0 if gg % 2 == 0 else s1
                if gg >= 2:
                    copies[gg % 2].wait()

                def zrow(r, _, buf=buf):
                    def zcol(jj, __):
                        for k in range(8):
                            buf[r, pl.ds((jj * 8 + k) * 16, 16)] = (
                                jnp.zeros((16,), jnp.float32))
                        return 0
                    lax.fori_loop(0, HW_COLS // 128, zcol, 0)
                    return 0
                lax.fori_loop(0, GRP, zrow, 0)

                def pbody(p, _, buf=buf, g=g):
                    xr = x_v[p, pl.ds(g * GRP, GRP)]
                    lr = lam_v[p]
                    nb = jnp.minimum((xr * 841.0).astype(jnp.int32), 840)
                    plsc.addupdate_scatter(buf, [lanes, nb], lr)
                    plsc.addupdate_scatter(buf, [lanes, nb + NBINS], lr * xr)
                    return 0
                lax.fori_loop(0, IN_DIM, pbody, 0)

                rowbase = colbase + g * GRP
                copies[gg % 2] = pltpu.async_copy(
                    buf,
                    out_hbm.at[pl.ds(pl.multiple_of(rowbase, 8), GRP), :],
                    sem)
                gg += 1
        copies[0].wait()
        copies[1].wait()

    pl.run_scoped(
        _inner,
        pltpu.VMEM((IN_DIM, 16), jnp.float32),
        pltpu.VMEM((IN_DIM, XCHUNK), jnp.float32),
        pltpu.VMEM((GRP, HW_COLS), jnp.float32),
        pltpu.VMEM((GRP, HW_COLS), jnp.float32),
        pltpu.SemaphoreType.DMA,
        pltpu.SemaphoreType.DMA,
    )


def _sc_hist(xt, lamb16):
    mesh = plsc.VectorSubcoreMesh(core_axis_name="c", subcore_axis_name="s")
    return pl.kernel(
        _sc_hist_body,
        mesh=mesh,
        out_type=jax.ShapeDtypeStruct((N_SAMPLES, HW_COLS), jnp.float32),
        compiler_params=pltpu.CompilerParams(needs_layout_passes=False),
    )(xt, lamb16)


def _tc_table_kernel(pli_ref, t_ref):
    inc = jax.nn.softplus(pli_ref[...])
    tot = jnp.sum(inc) + 1e-8
    c0 = inc[0, 0] / tot
    minv = 1.0 / (tot * DPHI)
    m = [inc[0, j + 1] * minv for j in range(NUM_KNOTS - 1)]
    A = {i: m[i - 1] - m[i] for i in range(1, NUM_KNOTS - 1)}
    A[NUM_KNOTS - 1] = m[NUM_KNOTS - 2]

    ni = lax.broadcasted_iota(jnp.int32, (NBINS, OUT_DIM), 0).astype(
        jnp.float32)
    qi = lax.broadcasted_iota(jnp.int32, (NBINS, OUT_DIM), 1).astype(
        jnp.float32)
    zf = qi * 29.0 + ni
    bacc = jnp.zeros((NBINS, OUT_DIM), jnp.float32)
    cacc = jnp.zeros((NBINS, OUT_DIM), jnp.float32)
    for i in range(1, NUM_KNOTS):
        hi = zf < float(92 * i)
        bacc = bacc + jnp.where(hi, A[i], 0.0)
        cacc = cacc + jnp.where(hi, 0.0, A[i] * float(i * DPHI))
    t_ref[0:NBINS, :] = cacc + c0 + (qi * ETA) * bacc
    t_ref[NBINS:HW_COLS, :] = bacc


def _tc_table(pli2):
    return pl.pallas_call(
        _tc_table_kernel,
        out_shape=jax.ShapeDtypeStruct((HW_COLS, OUT_DIM), jnp.float32),
    )(pli2)


def _tc_eval_kernel(hw_ref, t_ref, pc_ref, ccr_ref, out_ref):
    s = lax.dot_general(hw_ref[...], t_ref[...],
                        (((1,), (0,)), ((), ())),
                        precision=lax.Precision.HIGHEST,
                        preferred_element_type=jnp.float32)

    C = pc_ref[...]
    cmin = jnp.min(C)
    cmax = jnp.max(C)
    cc = ccr_ref[0, 0]
    cr = ccr_ref[0, 1]
    alpha = 2.0 * cr / (cmax - cmin + 1e-8)
    tc0 = cc - cr + alpha * (C[0, 0] - cmin)
    M2 = [alpha * (C[0, j + 1] - C[0, j]) / DPHI2 for j in range(NUM_KNOTS - 1)]
    A2 = [M2[i - 1] - M2[i] for i in range(1, NUM_KNOTS - 1)] + [M2[NUM_KNOTS - 2]]

    Sc = jnp.maximum(s, 0.0)
    y = A2[0] * jnp.minimum(Sc, DPHI2)
    for i in range(1, NUM_KNOTS - 1):
        y = y + A2[i] * jnp.minimum(Sc, (i + 1) * DPHI2)
    out_ref[...] = y + tc0


def _tc_eval(hw, tbl, pc2, ccr):
    grid = (N_SAMPLES // LANE_BLK,)
    return pl.pallas_call(
        _tc_eval_kernel,
        grid=grid,
        in_specs=[
            pl.BlockSpec((LANE_BLK, HW_COLS), lambda i: (i, 0)),
            pl.BlockSpec((HW_COLS, OUT_DIM), lambda i: (0, 0)),
            pl.BlockSpec((1, NUM_KNOTS), lambda i: (0, 0)),
            pl.BlockSpec((1, 2), lambda i: (0, 0)),
        ],
        out_specs=pl.BlockSpec((LANE_BLK, OUT_DIM), lambda i: (i, 0)),
        out_shape=jax.ShapeDtypeStruct((N_SAMPLES, OUT_DIM), jnp.float32),
    )(hw, tbl, pc2, ccr)


def kernel(x, phi_log_increments, Phi_coeffs, lambdas, cc, cr):
    xt = x.T
    pli2 = phi_log_increments.reshape(1, NUM_KNOTS)
    pc2 = Phi_coeffs.reshape(1, NUM_KNOTS)
    ccr = jnp.stack([jnp.asarray(cc, jnp.float32),
                     jnp.asarray(cr, jnp.float32)]).reshape(1, 2)
    lamb16 = jnp.broadcast_to(lambdas.reshape(IN_DIM, 1), (IN_DIM, 16))

    hw = _sc_hist(xt, lamb16)
    tbl = _tc_table(pli2)
    return _tc_eval(hw, tbl, pc2, ccr)


# speedup vs baseline: 2.4222x; 1.1643x over previous
import jax
import jax.numpy as jnp
from jax import lax
from jax.experimental import pallas as pl
from jax.experimental.pallas import tpu as pltpu
from jax.experimental.pallas import tpu_sc as plsc

NUM_KNOTS = 30
IN_DIM = 64
OUT_DIM = 64
N_SAMPLES = 8192
ETA = 1.0 / (NUM_KNOTS - 1)
PHI_MAX = 1.0 + (OUT_DIM - 1) * ETA
DPHI = PHI_MAX / (NUM_KNOTS - 1)
DPHI2 = 1.0 / (NUM_KNOTS - 1)
NBINS = 896
HW_COLS = 2 * NBINS
LANE_BLK = 256
NWORKERS = 32
ROWS_PER_W = N_SAMPLES // NWORKERS
XCHUNK = 128
GRP = 16


def _sc_hist_body(xt_hbm, lamb_hbm, out_hbm):
    def _inner(lam_v, x_v, h0, h1, s0, s1):
        wid = lax.axis_index("s") * 2 + lax.axis_index("c")
        lanes = lax.iota(jnp.int32, 16)
        pltpu.sync_copy(lamb_hbm, lam_v)

        copies = [None, None]
        gg = 0
        for blk in range(ROWS_PER_W // XCHUNK):
            colbase = wid * ROWS_PER_W + blk * XCHUNK
            pltpu.sync_copy(
                xt_hbm.at[:, pl.ds(pl.multiple_of(colbase, 128), XCHUNK)], x_v)
            for g in range(XCHUNK // GRP):
                buf = h0 if gg % 2 == 0 else h1
                sem = s0 if gg % 2 == 0 else s1
                if gg >= 2:
                    copies[gg % 2].wait()

                def zrow(r, _, buf=buf):
                    def zcol(jj, __):
                        for k in range(8):
                            buf[r, pl.ds((jj * 8 + k) * 16, 16)] = (
                                jnp.zeros((16,), jnp.float32))
                        return 0
                    lax.fori_loop(0, HW_COLS // 128, zcol, 0)
                    return 0
                lax.fori_loop(0, GRP, zrow, 0)

                def pbody(p, _, buf=buf, g=g):
                    xr = x_v[p, pl.ds(g * GRP, GRP)]
                    lr = lam_v[p]
                    nb = jnp.minimum((xr * 841.0).astype(jnp.int32), 840)
                    plsc.addupdate_scatter(buf, [lanes, nb], lr)
                    plsc.addupdate_scatter(buf, [lanes, nb + NBINS], lr * xr)
                    return 0
                lax.fori_loop(0, IN_DIM, pbody, 0)

                rowbase = colbase + g * GRP
                copies[gg % 2] = pltpu.async_copy(
                    buf,
                    out_hbm.at[pl.ds(pl.multiple_of(rowbase, 8), GRP), :],
                    sem)
                gg += 1
        copies[0].wait()
        copies[1].wait()

    pl.run_scoped(
        _inner,
        pltpu.VMEM((IN_DIM, 16), jnp.float32),
        pltpu.VMEM((IN_DIM, XCHUNK), jnp.float32),
        pltpu.VMEM((GRP, HW_COLS), jnp.float32),
        pltpu.VMEM((GRP, HW_COLS), jnp.float32),
        pltpu.SemaphoreType.DMA,
        pltpu.SemaphoreType.DMA,
    )


def _sc_hist(xt, lamb16):
    mesh = plsc.VectorSubcoreMesh(core_axis_name="c", subcore_axis_name="s")
    return pl.kernel(
        _sc_hist_body,
        mesh=mesh,
        out_type=jax.ShapeDtypeStruct((N_SAMPLES, HW_COLS), jnp.float32),
        compiler_params=pltpu.CompilerParams(needs_layout_passes=False),
    )(xt, lamb16)


def _tc_table_kernel(pli_ref, t_hi_ref, t_lo_ref):
    inc = jax.nn.softplus(pli_ref[...])
    tot = jnp.sum(inc) + 1e-8
    c0 = inc[0, 0] / tot
    minv = 1.0 / (tot * DPHI)
    m = [inc[0, j + 1] * minv for j in range(NUM_KNOTS - 1)]
    A = {i: m[i - 1] - m[i] for i in range(1, NUM_KNOTS - 1)}
    A[NUM_KNOTS - 1] = m[NUM_KNOTS - 2]

    ni = lax.broadcasted_iota(jnp.int32, (NBINS, OUT_DIM), 0).astype(
        jnp.float32)
    qi = lax.broadcasted_iota(jnp.int32, (NBINS, OUT_DIM), 1).astype(
        jnp.float32)
    zf = qi * 29.0 + ni
    bacc = jnp.zeros((NBINS, OUT_DIM), jnp.float32)
    cacc = jnp.zeros((NBINS, OUT_DIM), jnp.float32)
    for i in range(1, NUM_KNOTS):
        hi = zf < float(92 * i)
        bacc = bacc + jnp.where(hi, A[i], 0.0)
        cacc = cacc + jnp.where(hi, 0.0, A[i] * float(i * DPHI))
    tfull = jnp.concatenate([cacc + c0 + (qi * ETA) * bacc, bacc], axis=0)
    th = tfull.astype(jnp.bfloat16)
    t_hi_ref[...] = th
    t_lo_ref[...] = (tfull - th.astype(jnp.float32)).astype(jnp.bfloat16)


def _tc_table(pli2):
    return pl.pallas_call(
        _tc_table_kernel,
        out_shape=[jax.ShapeDtypeStruct((HW_COLS, OUT_DIM), jnp.bfloat16),
                   jax.ShapeDtypeStruct((HW_COLS, OUT_DIM), jnp.bfloat16)],
    )(pli2)


def _tc_eval_kernel(hw_ref, th_ref, tl_ref, pc_ref, ccr_ref, out_ref):
    hw = hw_ref[...]
    hwh = hw.astype(jnp.bfloat16)
    hwl = (hw - hwh.astype(jnp.float32)).astype(jnp.bfloat16)
    dims = (((1,), (0,)), ((), ()))
    s = (lax.dot_general(hwh, th_ref[...], dims,
                         preferred_element_type=jnp.float32)
         + lax.dot_general(hwh, tl_ref[...], dims,
                           preferred_element_type=jnp.float32)
         + lax.dot_general(hwl, th_ref[...], dims,
                           preferred_element_type=jnp.float32))

    C = pc_ref[...]
    cmin = jnp.min(C)
    cmax = jnp.max(C)
    cc = ccr_ref[0, 0]
    cr = ccr_ref[0, 1]
    alpha = 2.0 * cr / (cmax - cmin + 1e-8)
    tc0 = cc - cr + alpha * (C[0, 0] - cmin)
    M2 = [alpha * (C[0, j + 1] - C[0, j]) / DPHI2 for j in range(NUM_KNOTS - 1)]
    A2 = [M2[i - 1] - M2[i] for i in range(1, NUM_KNOTS - 1)] + [M2[NUM_KNOTS - 2]]

    Sc = jnp.maximum(s, 0.0)
    y = A2[0] * jnp.minimum(Sc, DPHI2)
    for i in range(1, NUM_KNOTS - 1):
        y = y + A2[i] * jnp.minimum(Sc, (i + 1) * DPHI2)
    out_ref[...] = y + tc0


def _tc_eval(hw, th, tl, pc2, ccr):
    grid = (N_SAMPLES // LANE_BLK,)
    return pl.pallas_call(
        _tc_eval_kernel,
        grid=grid,
        in_specs=[
            pl.BlockSpec((LANE_BLK, HW_COLS), lambda i: (i, 0)),
            pl.BlockSpec((HW_COLS, OUT_DIM), lambda i: (0, 0)),
            pl.BlockSpec((HW_COLS, OUT_DIM), lambda i: (0, 0)),
            pl.BlockSpec((1, NUM_KNOTS), lambda i: (0, 0)),
            pl.BlockSpec((1, 2), lambda i: (0, 0)),
        ],
        out_specs=pl.BlockSpec((LANE_BLK, OUT_DIM), lambda i: (i, 0)),
        out_shape=jax.ShapeDtypeStruct((N_SAMPLES, OUT_DIM), jnp.float32),
    )(hw, th, tl, pc2, ccr)


def kernel(x, phi_log_increments, Phi_coeffs, lambdas, cc, cr):
    xt = x.T
    pli2 = phi_log_increments.reshape(1, NUM_KNOTS)
    pc2 = Phi_coeffs.reshape(1, NUM_KNOTS)
    ccr = jnp.stack([jnp.asarray(cc, jnp.float32),
                     jnp.asarray(cr, jnp.float32)]).reshape(1, 2)
    lamb16 = jnp.broadcast_to(lambdas.reshape(IN_DIM, 1), (IN_DIM, 16))

    hw = _sc_hist(xt, lamb16)
    th, tl = _tc_table(pli2)
    return _tc_eval(hw, th, tl, pc2, ccr)


# speedup vs baseline: 2.6539x; 1.0957x over previous
import jax
import jax.numpy as jnp
from jax import lax
from jax.experimental import pallas as pl
from jax.experimental.pallas import tpu as pltpu
from jax.experimental.pallas import tpu_sc as plsc

NUM_KNOTS = 30
IN_DIM = 64
OUT_DIM = 64
N_SAMPLES = 8192
ETA = 1.0 / (NUM_KNOTS - 1)
PHI_MAX = 1.0 + (OUT_DIM - 1) * ETA
DPHI = PHI_MAX / (NUM_KNOTS - 1)
DPHI2 = 1.0 / (NUM_KNOTS - 1)
NBINS = 896
HW_COLS = 2 * NBINS
LANE_BLK = 512
NWORKERS = 32
ROWS_PER_W = N_SAMPLES // NWORKERS
XCHUNK = 256
GRP = 16


def _sc_hist_body(xt_hbm, lamb_hbm, out_hbm):
    def _inner(lam_v, x_v, h0, h1, s0, s1):
        wid = lax.axis_index("s") * 2 + lax.axis_index("c")
        lanes = lax.iota(jnp.int32, 16)
        pltpu.sync_copy(lamb_hbm, lam_v)

        copies = [None, None]
        gg = 0
        for blk in range(ROWS_PER_W // XCHUNK):
            colbase = wid * ROWS_PER_W + blk * XCHUNK
            pltpu.sync_copy(
                xt_hbm.at[:, pl.ds(pl.multiple_of(colbase, 128), XCHUNK)], x_v)
            for g in range(XCHUNK // GRP):
                buf = h0 if gg % 2 == 0 else h1
                sem = s0 if gg % 2 == 0 else s1
                if gg >= 2:
                    copies[gg % 2].wait()

                def zrow(r, _, buf=buf):
                    def zcol(jj, __):
                        for k in range(8):
                            buf[r, pl.ds((jj * 8 + k) * 16, 16)] = (
                                jnp.zeros((16,), jnp.float32))
                        return 0
                    lax.fori_loop(0, HW_COLS // 128, zcol, 0)
                    return 0
                lax.fori_loop(0, GRP, zrow, 0)

                def pbody(p, _, buf=buf, g=g):
                    xr = x_v[p, pl.ds(g * GRP, GRP)]
                    lr = lam_v[p]
                    nb = jnp.minimum((xr * 841.0).astype(jnp.int32), 840)
                    plsc.addupdate_scatter(buf, [lanes, nb], lr)
                    plsc.addupdate_scatter(buf, [lanes, nb + NBINS], lr * xr)
                    return 0
                lax.fori_loop(0, IN_DIM, pbody, 0)

                rowbase = colbase + g * GRP
                copies[gg % 2] = pltpu.async_copy(
                    buf,
                    out_hbm.at[pl.ds(pl.multiple_of(rowbase, 8), GRP), :],
                    sem)
                gg += 1
        copies[0].wait()
        copies[1].wait()

    pl.run_scoped(
        _inner,
        pltpu.VMEM((IN_DIM, 16), jnp.float32),
        pltpu.VMEM((IN_DIM, XCHUNK), jnp.float32),
        pltpu.VMEM((GRP, HW_COLS), jnp.float32),
        pltpu.VMEM((GRP, HW_COLS), jnp.float32),
        pltpu.SemaphoreType.DMA,
        pltpu.SemaphoreType.DMA,
    )


def _sc_hist(xt, lamb16):
    mesh = plsc.VectorSubcoreMesh(core_axis_name="c", subcore_axis_name="s")
    return pl.kernel(
        _sc_hist_body,
        mesh=mesh,
        out_type=jax.ShapeDtypeStruct((N_SAMPLES, HW_COLS), jnp.float32),
        compiler_params=pltpu.CompilerParams(needs_layout_passes=False),
    )(xt, lamb16)


def _tc_table_kernel(pli_ref, t_hi_ref, t_lo_ref):
    inc = jax.nn.softplus(pli_ref[...])
    tot = jnp.sum(inc) + 1e-8
    c0 = inc[0, 0] / tot
    minv = 1.0 / (tot * DPHI)
    m = [inc[0, j + 1] * minv for j in range(NUM_KNOTS - 1)]
    A = {i: m[i - 1] - m[i] for i in range(1, NUM_KNOTS - 1)}
    A[NUM_KNOTS - 1] = m[NUM_KNOTS - 2]

    ni = lax.broadcasted_iota(jnp.int32, (NBINS, OUT_DIM), 0).astype(
        jnp.float32)
    qi = lax.broadcasted_iota(jnp.int32, (NBINS, OUT_DIM), 1).astype(
        jnp.float32)
    zf = qi * 29.0 + ni
    bacc = jnp.zeros((NBINS, OUT_DIM), jnp.float32)
    cacc = jnp.zeros((NBINS, OUT_DIM), jnp.float32)
    for i in range(1, NUM_KNOTS):
        hi = zf < float(92 * i)
        bacc = bacc + jnp.where(hi, A[i], 0.0)
        cacc = cacc + jnp.where(hi, 0.0, A[i] * float(i * DPHI))
    tfull = jnp.concatenate([cacc + c0 + (qi * ETA) * bacc, bacc], axis=0)
    th = tfull.astype(jnp.bfloat16)
    t_hi_ref[...] = th
    t_lo_ref[...] = (tfull - th.astype(jnp.float32)).astype(jnp.bfloat16)


def _tc_table(pli2):
    return pl.pallas_call(
        _tc_table_kernel,
        out_shape=[jax.ShapeDtypeStruct((HW_COLS, OUT_DIM), jnp.bfloat16),
                   jax.ShapeDtypeStruct((HW_COLS, OUT_DIM), jnp.bfloat16)],
    )(pli2)


def _tc_eval_kernel(hw_ref, th_ref, tl_ref, pc_ref, ccr_ref, out_ref):
    hw = hw_ref[...]
    hwh = hw.astype(jnp.bfloat16)
    hwl = (hw - hwh.astype(jnp.float32)).astype(jnp.bfloat16)
    dims = (((1,), (0,)), ((), ()))
    s = (lax.dot_general(hwh, th_ref[...], dims,
                         preferred_element_type=jnp.float32)
         + lax.dot_general(hwh, tl_ref[...], dims,
                           preferred_element_type=jnp.float32)
         + lax.dot_general(hwl, th_ref[...], dims,
                           preferred_element_type=jnp.float32))

    C = pc_ref[...]
    cmin = jnp.min(C)
    cmax = jnp.max(C)
    cc = ccr_ref[0, 0]
    cr = ccr_ref[0, 1]
    alpha = 2.0 * cr / (cmax - cmin + 1e-8)
    tc0 = cc - cr + alpha * (C[0, 0] - cmin)
    M2 = [alpha * (C[0, j + 1] - C[0, j]) / DPHI2 for j in range(NUM_KNOTS - 1)]
    A2 = [M2[i - 1] - M2[i] for i in range(1, NUM_KNOTS - 1)] + [M2[NUM_KNOTS - 2]]

    Sc = jnp.maximum(s, 0.0)
    y = A2[0] * jnp.minimum(Sc, DPHI2)
    for i in range(1, NUM_KNOTS - 1):
        y = y + A2[i] * jnp.minimum(Sc, (i + 1) * DPHI2)
    out_ref[...] = y + tc0


def _tc_eval(hw, th, tl, pc2, ccr):
    grid = (N_SAMPLES // LANE_BLK,)
    return pl.pallas_call(
        _tc_eval_kernel,
        grid=grid,
        in_specs=[
            pl.BlockSpec((LANE_BLK, HW_COLS), lambda i: (i, 0)),
            pl.BlockSpec((HW_COLS, OUT_DIM), lambda i: (0, 0)),
            pl.BlockSpec((HW_COLS, OUT_DIM), lambda i: (0, 0)),
            pl.BlockSpec((1, NUM_KNOTS), lambda i: (0, 0)),
            pl.BlockSpec((1, 2), lambda i: (0, 0)),
        ],
        out_specs=pl.BlockSpec((LANE_BLK, OUT_DIM), lambda i: (i, 0)),
        out_shape=jax.ShapeDtypeStruct((N_SAMPLES, OUT_DIM), jnp.float32),
    )(hw, th, tl, pc2, ccr)


def kernel(x, phi_log_increments, Phi_coeffs, lambdas, cc, cr):
    xt = x.T
    pli2 = phi_log_increments.reshape(1, NUM_KNOTS)
    pc2 = Phi_coeffs.reshape(1, NUM_KNOTS)
    ccr = jnp.stack([jnp.asarray(cc, jnp.float32),
                     jnp.asarray(cr, jnp.float32)]).reshape(1, 2)
    lamb16 = jnp.broadcast_to(lambdas.reshape(IN_DIM, 1), (IN_DIM, 16))

    hw = _sc_hist(xt, lamb16)
    th, tl = _tc_table(pli2)
    return _tc_eval(hw, th, tl, pc2, ccr)


# speedup vs baseline: 2.6673x; 1.0050x over previous
import jax
import jax.numpy as jnp
from jax import lax
from jax.experimental import pallas as pl
from jax.experimental.pallas import tpu as pltpu
from jax.experimental.pallas import tpu_sc as plsc

NUM_KNOTS = 30
IN_DIM = 64
OUT_DIM = 64
N_SAMPLES = 8192
ETA = 1.0 / (NUM_KNOTS - 1)
PHI_MAX = 1.0 + (OUT_DIM - 1) * ETA
DPHI = PHI_MAX / (NUM_KNOTS - 1)
DPHI2 = 1.0 / (NUM_KNOTS - 1)
NBINS = 896
HW_COLS = 2 * NBINS
LANE_BLK = 1024
NWORKERS = 32
ROWS_PER_W = N_SAMPLES // NWORKERS
XCHUNK = 256
GRP = 16


def _sc_hist_body(xt_hbm, lamb_hbm, out_hbm):
    def _inner(lam_v, x_v, h0, h1, s0, s1):
        wid = lax.axis_index("s") * 2 + lax.axis_index("c")
        lanes = lax.iota(jnp.int32, 16)
        pltpu.sync_copy(lamb_hbm, lam_v)

        copies = [None, None]
        gg = 0
        for blk in range(ROWS_PER_W // XCHUNK):
            colbase = wid * ROWS_PER_W + blk * XCHUNK
            pltpu.sync_copy(
                xt_hbm.at[:, pl.ds(pl.multiple_of(colbase, 128), XCHUNK)], x_v)
            for g in range(XCHUNK // GRP):
                buf = h0 if gg % 2 == 0 else h1
                sem = s0 if gg % 2 == 0 else s1
                if gg >= 2:
                    copies[gg % 2].wait()

                def zrow(r, _, buf=buf):
                    def zcol(jj, __):
                        for k in range(8):
                            buf[r, pl.ds((jj * 8 + k) * 16, 16)] = (
                                jnp.zeros((16,), jnp.float32))
                        return 0
                    lax.fori_loop(0, HW_COLS // 128, zcol, 0)
                    return 0
                lax.fori_loop(0, GRP, zrow, 0)

                def pbody(p, _, buf=buf, g=g):
                    xr = x_v[p, pl.ds(g * GRP, GRP)]
                    lr = lam_v[p]
                    nb = jnp.minimum((xr * 841.0).astype(jnp.int32), 840)
                    plsc.addupdate_scatter(buf, [lanes, nb], lr)
                    plsc.addupdate_scatter(buf, [lanes, nb + NBINS], lr * xr)
                    return 0
                lax.fori_loop(0, IN_DIM, pbody, 0)

                rowbase = colbase + g * GRP
                copies[gg % 2] = pltpu.async_copy(
                    buf,
                    out_hbm.at[pl.ds(pl.multiple_of(rowbase, 8), GRP), :],
                    sem)
                gg += 1
        copies[0].wait()
        copies[1].wait()

    pl.run_scoped(
        _inner,
        pltpu.VMEM((IN_DIM, 16), jnp.float32),
        pltpu.VMEM((IN_DIM, XCHUNK), jnp.float32),
        pltpu.VMEM((GRP, HW_COLS), jnp.float32),
        pltpu.VMEM((GRP, HW_COLS), jnp.float32),
        pltpu.SemaphoreType.DMA,
        pltpu.SemaphoreType.DMA,
    )


def _sc_hist(xt, lamb16):
    mesh = plsc.VectorSubcoreMesh(core_axis_name="c", subcore_axis_name="s")
    return pl.kernel(
        _sc_hist_body,
        mesh=mesh,
        out_type=jax.ShapeDtypeStruct((N_SAMPLES, HW_COLS), jnp.float32),
        compiler_params=pltpu.CompilerParams(needs_layout_passes=False),
    )(xt, lamb16)


def _tc_table_kernel(pli_ref, t_hi_ref, t_lo_ref):
    inc = jax.nn.softplus(pli_ref[...])
    tot = jnp.sum(inc) + 1e-8
    c0 = inc[0, 0] / tot
    minv = 1.0 / (tot * DPHI)
    m = [inc[0, j + 1] * minv for j in range(NUM_KNOTS - 1)]
    A = {i: m[i - 1] - m[i] for i in range(1, NUM_KNOTS - 1)}
    A[NUM_KNOTS - 1] = m[NUM_KNOTS - 2]

    ni = lax.broadcasted_iota(jnp.int32, (NBINS, OUT_DIM), 0).astype(
        jnp.float32)
    qi = lax.broadcasted_iota(jnp.int32, (NBINS, OUT_DIM), 1).astype(
        jnp.float32)
    zf = qi * 29.0 + ni
    bacc = jnp.zeros((NBINS, OUT_DIM), jnp.float32)
    cacc = jnp.zeros((NBINS, OUT_DIM), jnp.float32)
    for i in range(1, NUM_KNOTS):
        hi = zf < float(92 * i)
        bacc = bacc + jnp.where(hi, A[i], 0.0)
        cacc = cacc + jnp.where(hi, 0.0, A[i] * float(i * DPHI))
    tfull = jnp.concatenate([cacc + c0 + (qi * ETA) * bacc, bacc], axis=0)
    th = tfull.astype(jnp.bfloat16)
    t_hi_ref[...] = th
    t_lo_ref[...] = (tfull - th.astype(jnp.float32)).astype(jnp.bfloat16)


def _tc_table(pli2):
    return pl.pallas_call(
        _tc_table_kernel,
        out_shape=[jax.ShapeDtypeStruct((HW_COLS, OUT_DIM), jnp.bfloat16),
                   jax.ShapeDtypeStruct((HW_COLS, OUT_DIM), jnp.bfloat16)],
    )(pli2)


def _tc_eval_kernel(hw_ref, th_ref, tl_ref, pc_ref, ccr_ref, out_ref):
    hw = hw_ref[...]
    hwh = hw.astype(jnp.bfloat16)
    hwl = (hw - hwh.astype(jnp.float32)).astype(jnp.bfloat16)
    dims = (((1,), (0,)), ((), ()))
    s = (lax.dot_general(hwh, th_ref[...], dims,
                         preferred_element_type=jnp.float32)
         + lax.dot_general(hwh, tl_ref[...], dims,
                           preferred_element_type=jnp.float32)
         + lax.dot_general(hwl, th_ref[...], dims,
                           preferred_element_type=jnp.float32))

    C = pc_ref[...]
    cmin = jnp.min(C)
    cmax = jnp.max(C)
    cc = ccr_ref[0, 0]
    cr = ccr_ref[0, 1]
    alpha = 2.0 * cr / (cmax - cmin + 1e-8)
    tc0 = cc - cr + alpha * (C[0, 0] - cmin)
    M2 = [alpha * (C[0, j + 1] - C[0, j]) / DPHI2 for j in range(NUM_KNOTS - 1)]
    A2 = [M2[i - 1] - M2[i] for i in range(1, NUM_KNOTS - 1)] + [M2[NUM_KNOTS - 2]]

    Sc = jnp.maximum(s, 0.0)
    y = A2[0] * jnp.minimum(Sc, DPHI2)
    for i in range(1, NUM_KNOTS - 1):
        y = y + A2[i] * jnp.minimum(Sc, (i + 1) * DPHI2)
    out_ref[...] = y + tc0


def _tc_eval(hw, th, tl, pc2, ccr):
    grid = (N_SAMPLES // LANE_BLK,)
    return pl.pallas_call(
        _tc_eval_kernel,
        grid=grid,
        in_specs=[
            pl.BlockSpec((LANE_BLK, HW_COLS), lambda i: (i, 0)),
            pl.BlockSpec((HW_COLS, OUT_DIM), lambda i: (0, 0)),
            pl.BlockSpec((HW_COLS, OUT_DIM), lambda i: (0, 0)),
            pl.BlockSpec((1, NUM_KNOTS), lambda i: (0, 0)),
            pl.BlockSpec((1, 2), lambda i: (0, 0)),
        ],
        out_specs=pl.BlockSpec((LANE_BLK, OUT_DIM), lambda i: (i, 0)),
        out_shape=jax.ShapeDtypeStruct((N_SAMPLES, OUT_DIM), jnp.float32),
    )(hw, th, tl, pc2, ccr)


def kernel(x, phi_log_increments, Phi_coeffs, lambdas, cc, cr):
    xt = x.T
    pli2 = phi_log_increments.reshape(1, NUM_KNOTS)
    pc2 = Phi_coeffs.reshape(1, NUM_KNOTS)
    ccr = jnp.stack([jnp.asarray(cc, jnp.float32),
                     jnp.asarray(cr, jnp.float32)]).reshape(1, 2)
    lamb16 = jnp.broadcast_to(lambdas.reshape(IN_DIM, 1), (IN_DIM, 16))

    hw = _sc_hist(xt, lamb16)
    th, tl = _tc_table(pli2)
    return _tc_eval(hw, th, tl, pc2, ccr)
